# Initial kernel scaffold; baseline (speedup 1.0000x reference)
import jax
import jax.numpy as jnp
from jax.experimental import pallas as pl


def kernel(x, edge_index, W_mlp, b_mlp, W_gcn, b_gcn, bn_gamma, bn_beta, W_last, b_last):
    raise NotImplementedError("write your pallas kernel here")


# speedup vs baseline: 8.3694x; 8.3694x over previous
import functools

import jax
import jax.numpy as jnp
from jax import lax
from jax.experimental import pallas as pl
from jax.experimental.pallas import tpu as pltpu
from jax.experimental.pallas import tpu_sc as plsc

N = 10000
E = 320000
R = 4
F = 128
OUT = 64
EPR = E // R

NC = 2
NS = 16
EPT = EPR // NS
CH = 128
NCH = 40
NPT = NCH * CH
NPAD = 10240
DUMMY = N
ZPT = NPAD // NS // 16
OPT = NPAD // NS

BLK = 1000
GRID = N // BLK

_mesh = functools.partial(
    plsc.VectorSubcoreMesh, core_axis_name="c", subcore_axis_name="s"
)


def _for_each_owned_relation(c, fn):
    for k in range(2):
        for core_id in range(NC):
            r = core_id * 2 + k

            @pl.when(c == core_id)
            def _(r=r):
                fn(r)


def _deg_body(dstp_hbm, out_hbm, didx, ones, zb, acc1, sem):
    del sem
    c = lax.axis_index("c")
    s = lax.axis_index("s")
    one16 = jnp.full((16,), 1.0, jnp.float32)
    zero16 = jnp.zeros((16,), jnp.float32)
    for i in range(CH):
        ones[i, :] = one16
    for i in range(16):
        zb[i, :] = zero16

    def run(r):
        def zbody(i, _):
            pltpu.sync_copy(zb, acc1.at[pl.ds(s * 640 + i * 16, 16)])
            return 0

        lax.fori_loop(0, ZPT, zbody, 0)
        pltpu.sync_copy(dstp_hbm.at[pl.ds((r * NS + s) * NCH, NCH)], didx)
        plsc.subcore_barrier()

        def body(j, _):
            pltpu.sync_copy(ones, acc1.at[didx.at[j]], add=True)
            return 0

        lax.fori_loop(0, NCH, body, 0)
        plsc.subcore_barrier()
        pltpu.sync_copy(
            acc1.at[pl.ds(s * OPT, OPT)],
            out_hbm.at[pl.ds(r * NPAD + s * OPT, OPT)],
        )
        plsc.subcore_barrier()

    _for_each_owned_relation(c, run)


def _deg_kernel(dst_pad):
    return pl.kernel(
        _deg_body,
        out_type=jax.ShapeDtypeStruct((R * NPAD, 16), jnp.float32),
        mesh=_mesh(),
        scratch_types=[
            pltpu.VMEM((NCH, CH), jnp.int32),
            pltpu.VMEM((CH, 16), jnp.float32),
            pltpu.VMEM((16, 16), jnp.float32),
            pltpu.VMEM_SHARED((NPAD, 16), jnp.float32),
            pltpu.SemaphoreType.DMA,
        ],
    )(dst_pad)


def _gs_body(tmp_hbm, srcp_hbm, dstp_hbm, out_hbm, sidx, didx, rows, zb, acc, sem):
    c = lax.axis_index("c")
    s = lax.axis_index("s")
    zero16 = jnp.zeros((16,), jnp.float32)
    for i in range(16):
        for j2 in range(F // 16):
            zb[i, pl.ds(j2 * 16, 16)] = zero16

    def run(r):
        def zbody(i, _):
            pltpu.sync_copy(zb, acc.at[pl.ds(s * 640 + i * 16, 16)])
            return 0

        lax.fori_loop(0, ZPT, zbody, 0)
        pltpu.sync_copy(srcp_hbm.at[pl.ds((r * NS + s) * NCH, NCH)], sidx)
        pltpu.sync_copy(dstp_hbm.at[pl.ds((r * NS + s) * NCH, NCH)], didx)
        plsc.subcore_barrier()

        def body(j, _):
            pltpu.async_copy(tmp_hbm.at[sidx.at[j]], rows, sem).wait()
            pltpu.sync_copy(rows, acc.at[didx.at[j]], add=True)
            return 0

        lax.fori_loop(0, NCH, body, 0)
        plsc.subcore_barrier()
        pltpu.sync_copy(
            acc.at[pl.ds(s * OPT, OPT)],
            out_hbm.at[pl.ds(r * NPAD + s * OPT, OPT)],
        )
        plsc.subcore_barrier()

    _for_each_owned_relation(c, run)


def _gs_kernel(tmp, src_pad, dst_pad):
    return pl.kernel(
        _gs_body,
        out_type=jax.ShapeDtypeStruct((R * NPAD, F), jnp.float32),
        mesh=_mesh(),
        scratch_types=[
            pltpu.VMEM((NCH, CH), jnp.int32),
            pltpu.VMEM((NCH, CH), jnp.int32),
            pltpu.VMEM((CH, F), jnp.float32),
            pltpu.VMEM((16, F), jnp.float32),
            pltpu.VMEM_SHARED((NPAD, F), jnp.float32),
            pltpu.SemaphoreType.DMA,
        ],
    )(tmp, src_pad, dst_pad)


def _tc1_body(x_ref, wm_ref, bm_ref, wg_ref, degc_ref, h_ref, dinv_ref, tmp_ref):
    h = jnp.maximum(
        jnp.dot(x_ref[...], wm_ref[...], preferred_element_type=jnp.float32)
        + bm_ref[...],
        0.0,
    )
    h_ref[...] = h
    deg = degc_ref[...][:, :, 0] + 1.0
    dinv = lax.rsqrt(deg)
    dinv_ref[...] = dinv
    for r in range(R):
        tmp_ref[r] = (
            jnp.dot(h, wg_ref[r], preferred_element_type=jnp.float32)
            * dinv[:, r : r + 1]
        )


def _tc1(x, w_mlp, b_mlp, wg0, degc):
    return pl.pallas_call(
        _tc1_body,
        grid=(GRID,),
        in_specs=[
            pl.BlockSpec((BLK, F), lambda i: (i, 0)),
            pl.BlockSpec((F, F), lambda i: (0, 0)),
            pl.BlockSpec((1, F), lambda i: (0, 0)),
            pl.BlockSpec((R, F, F), lambda i: (0, 0, 0)),
            pl.BlockSpec((BLK, R, 16), lambda i: (i, 0, 0)),
        ],
        out_specs=[
            pl.BlockSpec((BLK, F), lambda i: (i, 0)),
            pl.BlockSpec((BLK, R), lambda i: (i, 0)),
            pl.BlockSpec((R, BLK, F), lambda i: (0, i, 0)),
        ],
        out_shape=[
            jax.ShapeDtypeStruct((N, F), jnp.float32),
            jax.ShapeDtypeStruct((N, R), jnp.float32),
            jax.ShapeDtypeStruct((R, N, F), jnp.float32),
        ],
    )(x, w_mlp, b_mlp, wg0, degc)


def _post_body(acc_ref, tmp_ref, dinv_ref, bg_ref, h_ref, sums_ref):
    i = pl.program_id(0)
    acc = acc_ref[...]
    tmp = tmp_ref[...]
    dinv = dinv_ref[...]
    tot = jnp.zeros((BLK, F), jnp.float32)
    for r in range(R):
        tot = tot + (acc[r] + tmp[r]) * dinv[:, r : r + 1] + bg_ref[r]
    h = tot * (1.0 / R)
    h_ref[...] = h
    stats = jnp.concatenate(
        [
            jnp.sum(h, axis=0, keepdims=True),
            jnp.sum(h * h, axis=0, keepdims=True),
        ],
        axis=0,
    )

    @pl.when(i == 0)
    def _():
        sums_ref[...] = stats

    @pl.when(i > 0)
    def _():
        sums_ref[...] += stats


def _post(acc, tmp, dinv, bg):
    return pl.pallas_call(
        _post_body,
        grid=(GRID,),
        in_specs=[
            pl.BlockSpec((R, BLK, F), lambda i: (0, i, 0)),
            pl.BlockSpec((R, BLK, F), lambda i: (0, i, 0)),
            pl.BlockSpec((BLK, R), lambda i: (i, 0)),
            pl.BlockSpec((R, 1, F), lambda i: (0, 0, 0)),
        ],
        out_specs=[
            pl.BlockSpec((BLK, F), lambda i: (i, 0)),
            pl.BlockSpec((2, F), lambda i: (0, 0)),
        ],
        out_shape=[
            jax.ShapeDtypeStruct((N, F), jnp.float32),
            jax.ShapeDtypeStruct((2, F), jnp.float32),
        ],
    )(acc, tmp, dinv, bg)


def _bn_next_body(hpre_ref, sums_ref, g_ref, b_ref, wg_ref, dinv_ref, tmp_ref):
    mu = sums_ref[0:1, :] * (1.0 / N)
    ex2 = sums_ref[1:2, :] * (1.0 / N)
    var = ex2 - mu * mu
    scale = g_ref[...] * lax.rsqrt(var + 1e-5)
    h = jnp.maximum((hpre_ref[...] - mu) * scale + b_ref[...], 0.0)
    dinv = dinv_ref[...]
    for r in range(R):
        tmp_ref[r] = (
            jnp.dot(h, wg_ref[r], preferred_element_type=jnp.float32)
            * dinv[:, r : r + 1]
        )


def _bn_next(hpre, sums, gamma, beta, wg1, dinv):
    return pl.pallas_call(
        _bn_next_body,
        grid=(GRID,),
        in_specs=[
            pl.BlockSpec((BLK, F), lambda i: (i, 0)),
            pl.BlockSpec((2, F), lambda i: (0, 0)),
            pl.BlockSpec((1, F), lambda i: (0, 0)),
            pl.BlockSpec((1, F), lambda i: (0, 0)),
            pl.BlockSpec((R, F, F), lambda i: (0, 0, 0)),
            pl.BlockSpec((BLK, R), lambda i: (i, 0)),
        ],
        out_specs=[pl.BlockSpec((R, BLK, F), lambda i: (0, i, 0))],
        out_shape=[jax.ShapeDtypeStruct((R, N, F), jnp.float32)],
    )(hpre, sums, gamma, beta, wg1, dinv)[0]


def _final_body(hpre_ref, sums_ref, g_ref, b_ref, res_ref, wl_ref, bl_ref, out_ref):
    mu = sums_ref[0:1, :] * (1.0 / N)
    ex2 = sums_ref[1:2, :] * (1.0 / N)
    var = ex2 - mu * mu
    scale = g_ref[...] * lax.rsqrt(var + 1e-5)
    h = jnp.maximum((hpre_ref[...] - mu) * scale + b_ref[...], 0.0)
    h = h + res_ref[...]
    out_ref[...] = (
        jnp.dot(h, wl_ref[...], preferred_element_type=jnp.float32) + bl_ref[...]
    )


def _final(hpre, sums, gamma, beta, res, w_last, b_last):
    return pl.pallas_call(
        _final_body,
        grid=(GRID,),
        in_specs=[
            pl.BlockSpec((BLK, F), lambda i: (i, 0)),
            pl.BlockSpec((2, F), lambda i: (0, 0)),
            pl.BlockSpec((1, F), lambda i: (0, 0)),
            pl.BlockSpec((1, F), lambda i: (0, 0)),
            pl.BlockSpec((BLK, F), lambda i: (i, 0)),
            pl.BlockSpec((F, OUT), lambda i: (0, 0)),
            pl.BlockSpec((1, OUT), lambda i: (0, 0)),
        ],
        out_specs=[pl.BlockSpec((BLK, OUT), lambda i: (i, 0))],
        out_shape=[jax.ShapeDtypeStruct((N, OUT), jnp.float32)],
    )(hpre, sums, gamma, beta, res, w_last, b_last)[0]


def kernel(x, edge_index, W_mlp, b_mlp, W_gcn, b_gcn, bn_gamma, bn_beta, W_last, b_last):
    ei = edge_index.astype(jnp.int32)
    src = ei[0].reshape(R, NS, EPT)
    dst = ei[1].reshape(R, NS, EPT)
    src = src + (jnp.arange(R, dtype=jnp.int32) * N)[:, None, None]
    pad = ((0, 0), (0, 0), (0, NPT - EPT))
    src_pad = jnp.pad(src, pad, constant_values=0).reshape(R * NS * NCH, CH)
    dst_pad = jnp.pad(dst, pad, constant_values=DUMMY).reshape(R * NS * NCH, CH)

    degc = _deg_kernel(dst_pad).reshape(R, NPAD, 16)[:, :N].transpose(1, 0, 2)

    b_mlp2 = b_mlp.reshape(1, F)
    bg = b_gcn.reshape(L_SHAPE := (2, R, 1, F))
    gamma = bn_gamma.reshape(2, 1, F)
    beta = bn_beta.reshape(2, 1, F)

    h0, dinv, tmp = _tc1(x, W_mlp, b_mlp2, W_gcn[0], degc)

    acc = _gs_kernel(tmp.reshape(R * N, F), src_pad, dst_pad).reshape(R, NPAD, F)[:, :N]
    h1pre, sums1 = _post(acc, tmp, dinv, bg[0])
    tmp2 = _bn_next(h1pre, sums1, gamma[0], beta[0], W_gcn[1], dinv)

    acc2 = _gs_kernel(tmp2.reshape(R * N, F), src_pad, dst_pad).reshape(R, NPAD, F)[:, :N]
    h2pre, sums2 = _post(acc2, tmp2, dinv, bg[1])
    return _final(
        h2pre, sums2, gamma[1], beta[1], h0, W_last, b_last.reshape(1, OUT)
    )


# speedup vs baseline: 9.4186x; 1.1254x over previous
import functools

import jax
import jax.numpy as jnp
from jax import lax
from jax.experimental import pallas as pl
from jax.experimental.pallas import tpu as pltpu
from jax.experimental.pallas import tpu_sc as plsc

N = 10000
E = 320000
R = 4
F = 128
OUT = 64
EPR = E // R

NC = 2
NS = 16
EPT = EPR // NS
CH = 128
NCH = 40
NPT = NCH * CH
NPAD = 10240
DUMMY = N
ZPT = NPAD // NS // 16
OPT = NPAD // NS

BLK = 1000
GRID = N // BLK

_mesh = functools.partial(
    plsc.VectorSubcoreMesh, core_axis_name="c", subcore_axis_name="s"
)


def _for_each_owned_relation(c, fn):
    for k in range(2):
        for core_id in range(NC):
            r = core_id * 2 + k

            @pl.when(c == core_id)
            def _(r=r):
                fn(r)


def _deg_body(dstp_hbm, out_hbm, didx, ones, zb, acc1, sem):
    del sem
    c = lax.axis_index("c")
    s = lax.axis_index("s")
    one16 = jnp.full((16,), 1.0, jnp.float32)
    zero16 = jnp.zeros((16,), jnp.float32)
    for i in range(CH):
        ones[i, :] = one16
    for i in range(16):
        zb[i, :] = zero16

    def run(r):
        def zbody(i, _):
            pltpu.sync_copy(zb, acc1.at[pl.ds(s * 640 + i * 16, 16)])
            return 0

        lax.fori_loop(0, ZPT, zbody, 0)
        pltpu.sync_copy(dstp_hbm.at[pl.ds((r * NS + s) * NCH, NCH)], didx)
        plsc.subcore_barrier()

        def body(j, _):
            pltpu.sync_copy(ones, acc1.at[didx.at[j]], add=True)
            return 0

        lax.fori_loop(0, NCH, body, 0)
        plsc.subcore_barrier()
        pltpu.sync_copy(
            acc1.at[pl.ds(s * OPT, OPT)],
            out_hbm.at[pl.ds(r * NPAD + s * OPT, OPT)],
        )
        plsc.subcore_barrier()

    _for_each_owned_relation(c, run)


def _deg_kernel(dst_pad):
    return pl.kernel(
        _deg_body,
        out_type=jax.ShapeDtypeStruct((R * NPAD, 16), jnp.float32),
        mesh=_mesh(),
        scratch_types=[
            pltpu.VMEM((NCH, CH), jnp.int32),
            pltpu.VMEM((CH, 16), jnp.float32),
            pltpu.VMEM((16, 16), jnp.float32),
            pltpu.VMEM_SHARED((NPAD, 16), jnp.float32),
            pltpu.SemaphoreType.DMA,
        ],
    )(dst_pad)


def _gs_body(
    tmp_hbm, srcp_hbm, dstp_hbm, out_hbm, sidx, didx, rows0, rows1, acc, sem0, sem1
):
    c = lax.axis_index("c")
    s = lax.axis_index("s")
    zero16 = jnp.zeros((16,), jnp.float32)

    def _gather(j, rows, sem):
        pltpu.make_async_copy(tmp_hbm.at[sidx.at[j]], rows, sem).start()

    def _gwait(j, rows, sem):
        pltpu.make_async_copy(tmp_hbm.at[sidx.at[j]], rows, sem).wait()

    def run(r):
        def zfill(i, _):
            for j2 in range(F // 16):
                rows0[i, pl.ds(j2 * 16, 16)] = zero16
            return 0

        lax.fori_loop(0, CH, zfill, 0)

        def zbody(i, _):
            pltpu.sync_copy(rows0, acc.at[pl.ds(s * OPT + i * CH, CH)])
            return 0

        lax.fori_loop(0, OPT // CH, zbody, 0)
        pltpu.sync_copy(srcp_hbm.at[pl.ds((r * NS + s) * NCH, NCH)], sidx)
        pltpu.sync_copy(dstp_hbm.at[pl.ds((r * NS + s) * NCH, NCH)], didx)
        plsc.subcore_barrier()

        _gather(0, rows0, sem0)

        def body(i, _):
            _gather(2 * i + 1, rows1, sem1)
            _gwait(2 * i, rows0, sem0)
            pltpu.sync_copy(rows0, acc.at[didx.at[2 * i]], add=True)
            _gather(2 * i + 2, rows0, sem0)
            _gwait(2 * i + 1, rows1, sem1)
            pltpu.sync_copy(rows1, acc.at[didx.at[2 * i + 1]], add=True)
            return 0

        lax.fori_loop(0, NCH // 2 - 1, body, 0)
        _gather(NCH - 1, rows1, sem1)
        _gwait(NCH - 2, rows0, sem0)
        pltpu.sync_copy(rows0, acc.at[didx.at[NCH - 2]], add=True)
        _gwait(NCH - 1, rows1, sem1)
        pltpu.sync_copy(rows1, acc.at[didx.at[NCH - 1]], add=True)

        plsc.subcore_barrier()
        pltpu.sync_copy(
            acc.at[pl.ds(s * OPT, OPT)],
            out_hbm.at[pl.ds(r * NPAD + s * OPT, OPT)],
        )
        plsc.subcore_barrier()

    _for_each_owned_relation(c, run)


def _gs_kernel(tmp, src_pad, dst_pad):
    return pl.kernel(
        _gs_body,
        out_type=jax.ShapeDtypeStruct((R * NPAD, F), jnp.float32),
        mesh=_mesh(),
        scratch_types=[
            pltpu.VMEM((NCH, CH), jnp.int32),
            pltpu.VMEM((NCH, CH), jnp.int32),
            pltpu.VMEM((CH, F), jnp.float32),
            pltpu.VMEM((CH, F), jnp.float32),
            pltpu.VMEM_SHARED((NPAD, F), jnp.float32),
            pltpu.SemaphoreType.DMA,
            pltpu.SemaphoreType.DMA,
        ],
    )(tmp, src_pad, dst_pad)


def _tc1_body(x_ref, wm_ref, bm_ref, wg_ref, degc_ref, h_ref, dinv_ref, tmp_ref):
    h = jnp.maximum(
        jnp.dot(x_ref[...], wm_ref[...], preferred_element_type=jnp.float32)
        + bm_ref[...],
        0.0,
    )
    h_ref[...] = h
    deg = degc_ref[...][:, :, 0] + 1.0
    dinv = lax.rsqrt(deg)
    dinv_ref[...] = dinv
    for r in range(R):
        tmp_ref[r] = (
            jnp.dot(h, wg_ref[r], preferred_element_type=jnp.float32)
            * dinv[:, r : r + 1]
        )


def _tc1(x, w_mlp, b_mlp, wg0, degc):
    return pl.pallas_call(
        _tc1_body,
        grid=(GRID,),
        in_specs=[
            pl.BlockSpec((BLK, F), lambda i: (i, 0)),
            pl.BlockSpec((F, F), lambda i: (0, 0)),
            pl.BlockSpec((1, F), lambda i: (0, 0)),
            pl.BlockSpec((R, F, F), lambda i: (0, 0, 0)),
            pl.BlockSpec((BLK, R, 16), lambda i: (i, 0, 0)),
        ],
        out_specs=[
            pl.BlockSpec((BLK, F), lambda i: (i, 0)),
            pl.BlockSpec((BLK, R), lambda i: (i, 0)),
            pl.BlockSpec((R, BLK, F), lambda i: (0, i, 0)),
        ],
        out_shape=[
            jax.ShapeDtypeStruct((N, F), jnp.float32),
            jax.ShapeDtypeStruct((N, R), jnp.float32),
            jax.ShapeDtypeStruct((R, N, F), jnp.float32),
        ],
    )(x, w_mlp, b_mlp, wg0, degc)


def _post_body(acc_ref, tmp_ref, dinv_ref, bg_ref, h_ref, sums_ref):
    i = pl.program_id(0)
    acc = acc_ref[...]
    tmp = tmp_ref[...]
    dinv = dinv_ref[...]
    tot = jnp.zeros((BLK, F), jnp.float32)
    for r in range(R):
        tot = tot + (acc[r] + tmp[r]) * dinv[:, r : r + 1] + bg_ref[r]
    h = tot * (1.0 / R)
    h_ref[...] = h
    stats = jnp.concatenate(
        [
            jnp.sum(h, axis=0, keepdims=True),
            jnp.sum(h * h, axis=0, keepdims=True),
        ],
        axis=0,
    )

    @pl.when(i == 0)
    def _():
        sums_ref[...] = stats

    @pl.when(i > 0)
    def _():
        sums_ref[...] += stats


def _post(acc, tmp, dinv, bg):
    return pl.pallas_call(
        _post_body,
        grid=(GRID,),
        in_specs=[
            pl.BlockSpec((R, BLK, F), lambda i: (0, i, 0)),
            pl.BlockSpec((R, BLK, F), lambda i: (0, i, 0)),
            pl.BlockSpec((BLK, R), lambda i: (i, 0)),
            pl.BlockSpec((R, 1, F), lambda i: (0, 0, 0)),
        ],
        out_specs=[
            pl.BlockSpec((BLK, F), lambda i: (i, 0)),
            pl.BlockSpec((2, F), lambda i: (0, 0)),
        ],
        out_shape=[
            jax.ShapeDtypeStruct((N, F), jnp.float32),
            jax.ShapeDtypeStruct((2, F), jnp.float32),
        ],
    )(acc, tmp, dinv, bg)


def _bn_next_body(hpre_ref, sums_ref, g_ref, b_ref, wg_ref, dinv_ref, tmp_ref):
    mu = sums_ref[0:1, :] * (1.0 / N)
    ex2 = sums_ref[1:2, :] * (1.0 / N)
    var = ex2 - mu * mu
    scale = g_ref[...] * lax.rsqrt(var + 1e-5)
    h = jnp.maximum((hpre_ref[...] - mu) * scale + b_ref[...], 0.0)
    dinv = dinv_ref[...]
    for r in range(R):
        tmp_ref[r] = (
            jnp.dot(h, wg_ref[r], preferred_element_type=jnp.float32)
            * dinv[:, r : r + 1]
        )


def _bn_next(hpre, sums, gamma, beta, wg1, dinv):
    return pl.pallas_call(
        _bn_next_body,
        grid=(GRID,),
        in_specs=[
            pl.BlockSpec((BLK, F), lambda i: (i, 0)),
            pl.BlockSpec((2, F), lambda i: (0, 0)),
            pl.BlockSpec((1, F), lambda i: (0, 0)),
            pl.BlockSpec((1, F), lambda i: (0, 0)),
            pl.BlockSpec((R, F, F), lambda i: (0, 0, 0)),
            pl.BlockSpec((BLK, R), lambda i: (i, 0)),
        ],
        out_specs=[pl.BlockSpec((R, BLK, F), lambda i: (0, i, 0))],
        out_shape=[jax.ShapeDtypeStruct((R, N, F), jnp.float32)],
    )(hpre, sums, gamma, beta, wg1, dinv)[0]


def _final_body(hpre_ref, sums_ref, g_ref, b_ref, res_ref, wl_ref, bl_ref, out_ref):
    mu = sums_ref[0:1, :] * (1.0 / N)
    ex2 = sums_ref[1:2, :] * (1.0 / N)
    var = ex2 - mu * mu
    scale = g_ref[...] * lax.rsqrt(var + 1e-5)
    h = jnp.maximum((hpre_ref[...] - mu) * scale + b_ref[...], 0.0)
    h = h + res_ref[...]
    out_ref[...] = (
        jnp.dot(h, wl_ref[...], preferred_element_type=jnp.float32) + bl_ref[...]
    )


def _final(hpre, sums, gamma, beta, res, w_last, b_last):
    return pl.pallas_call(
        _final_body,
        grid=(GRID,),
        in_specs=[
            pl.BlockSpec((BLK, F), lambda i: (i, 0)),
            pl.BlockSpec((2, F), lambda i: (0, 0)),
            pl.BlockSpec((1, F), lambda i: (0, 0)),
            pl.BlockSpec((1, F), lambda i: (0, 0)),
            pl.BlockSpec((BLK, F), lambda i: (i, 0)),
            pl.BlockSpec((F, OUT), lambda i: (0, 0)),
            pl.BlockSpec((1, OUT), lambda i: (0, 0)),
        ],
        out_specs=[pl.BlockSpec((BLK, OUT), lambda i: (i, 0))],
        out_shape=[jax.ShapeDtypeStruct((N, OUT), jnp.float32)],
    )(hpre, sums, gamma, beta, res, w_last, b_last)[0]


def kernel(x, edge_index, W_mlp, b_mlp, W_gcn, b_gcn, bn_gamma, bn_beta, W_last, b_last):
    ei = edge_index.astype(jnp.int32)
    src = ei[0].reshape(R, NS, EPT)
    dst = ei[1].reshape(R, NS, EPT)
    src = src + (jnp.arange(R, dtype=jnp.int32) * N)[:, None, None]
    pad = ((0, 0), (0, 0), (0, NPT - EPT))
    src_pad = jnp.pad(src, pad, constant_values=0).reshape(R * NS * NCH, CH)
    dst_pad = jnp.pad(dst, pad, constant_values=DUMMY).reshape(R * NS * NCH, CH)

    degc = _deg_kernel(dst_pad).reshape(R, NPAD, 16)[:, :N].transpose(1, 0, 2)

    b_mlp2 = b_mlp.reshape(1, F)
    bg = b_gcn.reshape(L_SHAPE := (2, R, 1, F))
    gamma = bn_gamma.reshape(2, 1, F)
    beta = bn_beta.reshape(2, 1, F)

    h0, dinv, tmp = _tc1(x, W_mlp, b_mlp2, W_gcn[0], degc)

    acc = _gs_kernel(tmp.reshape(R * N, F), src_pad, dst_pad).reshape(R, NPAD, F)[:, :N]
    h1pre, sums1 = _post(acc, tmp, dinv, bg[0])
    tmp2 = _bn_next(h1pre, sums1, gamma[0], beta[0], W_gcn[1], dinv)

    acc2 = _gs_kernel(tmp2.reshape(R * N, F), src_pad, dst_pad).reshape(R, NPAD, F)[:, :N]
    h2pre, sums2 = _post(acc2, tmp2, dinv, bg[1])
    return _final(
        h2pre, sums2, gamma[1], beta[1], h0, W_last, b_last.reshape(1, OUT)
    )
